# initial kernel scaffold (unmeasured)
import jax
import jax.numpy as jnp
from jax import lax
from jax.experimental import pallas as pl
from jax.experimental.pallas import tpu as pltpu

Z = 4
B = 16
H = 16
D = 64
BS = 16
NSLOT = 128
NPG = 512
NPL = NPG // Z
NKL = NPL * BS
CW = D + 2

_NEG = -1e30


def _body(q_ref, k_ref, v_ref, bt_ref, lens_ref, out_ref,
          comm_ref, send_sems, recv_sems):
    my_x = lax.axis_index("x")
    my_y = lax.axis_index("y")
    my_z = lax.axis_index("z")

    barrier = pltpu.get_barrier_semaphore()
    for dz in range(1, Z):
        pl.semaphore_signal(
            barrier, inc=1,
            device_id=(my_x, my_y, (my_z + dz) % Z),
            device_id_type=pl.DeviceIdType.MESH,
        )
    pl.semaphore_wait(barrier, Z - 1)

    q = q_ref[:]
    k = k_ref[:]
    v = v_ref[:]

    pid = my_z * NPL + lax.broadcasted_iota(jnp.int32, (1, 1, NPL), 2)
    j = lax.broadcasted_iota(jnp.int32, (B, NSLOT, 1), 1)
    valid = j < lens_ref[:].reshape(B, 1, 1)
    match = (bt_ref[:].reshape(B, NSLOT, 1) == pid) & valid
    counts = jnp.sum(match.astype(jnp.float32), axis=1)
    w = jnp.broadcast_to(counts[:, :, None], (B, NPL, BS)).reshape(B, NKL)

    s = jnp.einsum("bhd,khd->bhk", q, k,
                   preferred_element_type=jnp.float32) * (D ** -0.5)
    wk = w[:, None, :]
    m = jnp.max(jnp.where(wk > 0, s, _NEG), axis=-1)
    m_safe = jnp.where(m < -1e29, 0.0, m)
    e = jnp.exp(s - m_safe[:, :, None]) * wk
    lsum = jnp.sum(e, axis=-1)
    u = jnp.einsum("bhk,khd->bhd", e, v,
                   preferred_element_type=jnp.float32)

    comm_ref[my_z, :, :, 0:D] = u
    comm_ref[my_z, :, :, D:D + 1] = m[:, :, None]
    comm_ref[my_z, :, :, D + 1:D + 2] = lsum[:, :, None]

    sends = []
    for dz in range(1, Z):
        rdma = pltpu.make_async_remote_copy(
            src_ref=comm_ref.at[my_z],
            dst_ref=comm_ref.at[my_z],
            send_sem=send_sems.at[dz - 1],
            recv_sem=recv_sems.at[my_z],
            device_id=(my_x, my_y, (my_z + dz) % Z),
            device_id_type=pl.DeviceIdType.MESH,
        )
        rdma.start()
        sends.append(rdma)

    for dz in range(1, Z):
        src_z = (my_z + dz) % Z
        recv = pltpu.make_async_remote_copy(
            src_ref=comm_ref.at[src_z],
            dst_ref=comm_ref.at[src_z],
            send_sem=send_sems.at[dz - 1],
            recv_sem=recv_sems.at[src_z],
            device_id=(my_x, my_y, src_z),
            device_id_type=pl.DeviceIdType.MESH,
        )
        recv.wait_recv()

    u_all = comm_ref[:, :, :, 0:D]
    m_all = comm_ref[:, :, :, D:D + 1]
    l_all = comm_ref[:, :, :, D + 1:D + 2]
    mg = jnp.max(m_all, axis=0)
    sc = jnp.exp(m_all - mg)
    big_l = jnp.sum(sc * l_all, axis=0)
    big_u = jnp.sum(sc * u_all, axis=0)
    out_ref[:, :, :] = big_u / big_l

    for rdma in sends:
        rdma.wait_send()


def kernel(Q, K, V, bt, lens):
    q3 = Q.reshape(B, H, D)
    k3 = K.reshape(NKL, H, D)
    v3 = V.reshape(NKL, H, D)
    lens2 = lens.reshape(B, 1)

    out = pl.pallas_call(
        _body,
        out_shape=jax.ShapeDtypeStruct((B, H, D), jnp.float32),
        in_specs=[pl.BlockSpec(memory_space=pltpu.VMEM)] * 5,
        out_specs=pl.BlockSpec(memory_space=pltpu.VMEM),
        scratch_shapes=[
            pltpu.VMEM((Z, B, H, CW), jnp.float32),
            pltpu.SemaphoreType.DMA((Z - 1,)),
            pltpu.SemaphoreType.DMA((Z,)),
        ],
        compiler_params=pltpu.CompilerParams(collective_id=0),
    )(q3, k3, v3, bt, lens2)
    return out.reshape(B, 1, H, D)


# baseline (device time: 68215 ns/iter reference)
import jax
import jax.numpy as jnp
from jax import lax
from jax.experimental import pallas as pl
from jax.experimental.pallas import tpu as pltpu

Z = 4
B = 16
H = 16
D = 64
BS = 16
NSLOT = 128
NPG = 512
NPL = NPG // Z
NKL = NPL * BS
CW = D + 2

_NEG = -1e30


def _body(q_ref, k_ref, v_ref, bt_ref, lens_ref, out_ref,
          comm_ref, send_sems, recv_sems):
    my_x = lax.axis_index("x")
    my_y = lax.axis_index("y")
    my_z = lax.axis_index("z")

    barrier = pltpu.get_barrier_semaphore()
    for dz in range(1, Z):
        pl.semaphore_signal(
            barrier, inc=1,
            device_id=(my_x, my_y, (my_z + dz) % Z),
            device_id_type=pl.DeviceIdType.MESH,
        )
    pl.semaphore_wait(barrier, Z - 1)

    pid = my_z * NPL + lax.broadcasted_iota(jnp.int32, (1, 1, NPL), 2)
    j = lax.broadcasted_iota(jnp.int32, (B, NSLOT, 1), 1)
    valid = j < lens_ref[:].reshape(B, 1, 1)
    match = (bt_ref[:].reshape(B, NSLOT, 1) == pid) & valid
    counts = jnp.sum(match.astype(jnp.float32), axis=1)
    w = jnp.broadcast_to(counts[:, :, None], (B, NPL, BS)).reshape(B, NKL)
    w_pos = w > 0

    scale = D ** -0.5
    for h in range(H):
        qh = q_ref[:, h * D:(h + 1) * D]
        kh = k_ref[:, h * D:(h + 1) * D]
        vh = v_ref[:, h * D:(h + 1) * D]
        s = lax.dot_general(
            qh, kh, (((1,), (1,)), ((), ())),
            preferred_element_type=jnp.float32,
        ) * scale
        m = jnp.max(jnp.where(w_pos, s, _NEG), axis=-1)
        m_safe = jnp.where(m < -1e29, 0.0, m)
        e = jnp.exp(s - m_safe[:, None]) * w
        lsum = jnp.sum(e, axis=-1)
        u = lax.dot_general(
            e, vh, (((1,), (0,)), ((), ())),
            preferred_element_type=jnp.float32,
        )
        comm_ref[my_z, :, h, 0:D] = u
        comm_ref[my_z, :, h, D:D + 1] = m[:, None]
        comm_ref[my_z, :, h, D + 1:D + 2] = lsum[:, None]

    sends = []
    for dz in range(1, Z):
        rdma = pltpu.make_async_remote_copy(
            src_ref=comm_ref.at[my_z],
            dst_ref=comm_ref.at[my_z],
            send_sem=send_sems.at[dz - 1],
            recv_sem=recv_sems.at[my_z],
            device_id=(my_x, my_y, (my_z + dz) % Z),
            device_id_type=pl.DeviceIdType.MESH,
        )
        rdma.start()
        sends.append(rdma)

    for dz in range(1, Z):
        src_z = (my_z + dz) % Z
        recv = pltpu.make_async_remote_copy(
            src_ref=comm_ref.at[src_z],
            dst_ref=comm_ref.at[src_z],
            send_sem=send_sems.at[dz - 1],
            recv_sem=recv_sems.at[src_z],
            device_id=(my_x, my_y, src_z),
            device_id_type=pl.DeviceIdType.MESH,
        )
        recv.wait_recv()

    u_all = comm_ref[:, :, :, 0:D]
    m_all = comm_ref[:, :, :, D:D + 1]
    l_all = comm_ref[:, :, :, D + 1:D + 2]
    mg = jnp.max(m_all, axis=0)
    sc = jnp.exp(m_all - mg)
    big_l = jnp.sum(sc * l_all, axis=0)
    big_u = jnp.sum(sc * u_all, axis=0)
    out_ref[:, :, :] = big_u / big_l

    for rdma in sends:
        rdma.wait_send()


def kernel(Q, K, V, bt, lens):
    q3 = Q.reshape(B, H * D)
    k3 = K.reshape(NKL, H * D)
    v3 = V.reshape(NKL, H * D)
    lens2 = lens.reshape(B, 1)

    out = pl.pallas_call(
        _body,
        out_shape=jax.ShapeDtypeStruct((B, H, D), jnp.float32),
        in_specs=[pl.BlockSpec(memory_space=pltpu.VMEM)] * 5,
        out_specs=pl.BlockSpec(memory_space=pltpu.VMEM),
        scratch_shapes=[
            pltpu.VMEM((Z, B, H, CW), jnp.float32),
            pltpu.SemaphoreType.DMA((Z - 1,)),
            pltpu.SemaphoreType.DMA((Z,)),
        ],
        compiler_params=pltpu.CompilerParams(collective_id=0),
    )(q3, k3, v3, bt, lens2)
    return out.reshape(B, 1, H, D)
